# Initial kernel scaffold; baseline (speedup 1.0000x reference)
import jax
import jax.numpy as jnp
from jax.experimental import pallas as pl


def kernel(inputs, edge_index, table, weight, bias):
    raise NotImplementedError("write your pallas kernel here")


# speedup vs baseline: 3.8837x; 3.8837x over previous
import jax
import jax.numpy as jnp
from jax import lax
from jax.experimental import pallas as pl
from jax.experimental.pallas import tpu as pltpu
from jax.experimental.pallas import tpu_sc as plsc

NC, NS, L = 2, 16, 16
NW = NC * NS
N_NODES = 10000
D = 128
N_EDGES = 320000
B = 4096

E_PER_W = N_EDGES // NW
ECHUNK = 80
N_ECH = E_PER_W // ECHUNK
RBLK = 16
N_RBLK = N_NODES // RBLK
B_PER_W = B // NW

_mesh = plsc.VectorSubcoreMesh(
    core_axis_name="c", subcore_axis_name="s", num_cores=NC, num_subcores=NS)

_f32 = jnp.float32


def _fill(buf, rows, value):
  v = jnp.full((L,), value, _f32)
  for r in range(rows):
    for q in range(D // L):
      buf[r, pl.ds(q * L, L)] = v


def _sliced_loop(s, body):
  def step(k, _):
    blk = s + NS * k
    @pl.when(blk < N_RBLK)
    def _():
      body(blk)
    return _
  lax.fori_loop(0, (N_RBLK + NS - 1) // NS, step, None)


def _zero_acc(s, acc, zbuf):
  _sliced_loop(s, lambda blk: pltpu.sync_copy(
      zbuf, acc.at[pl.ds(blk * RBLK, RBLK)]))


def _wb_acc(c, s, acc, hbm0, hbm1):
  def wb(blk):
    sl = pl.ds(blk * RBLK, RBLK)
    @pl.when(c == 0)
    def _():
      pltpu.sync_copy(acc.at[sl], hbm0.at[sl])
    @pl.when(c == 1)
    def _():
      pltpu.sync_copy(acc.at[sl], hbm1.at[sl])
  _sliced_loop(s, wb)


def _hop1_body(src_hbm, dst_hbm, tab_hbm,
               p0_hbm, p1_hbm, d0_hbm, d1_hbm,
               acc, idx_v, dst_v, rows_v, ones_v, zbuf, sem):
  c = lax.axis_index("c")
  s = lax.axis_index("s")
  wid = s * NC + c
  base = wid * E_PER_W

  _fill(zbuf, RBLK, 0.0)
  _fill(ones_v, ECHUNK, 1.0)

  _zero_acc(s, acc, zbuf)
  plsc.subcore_barrier()

  def deg_chunk(j, _):
    off = base + j * ECHUNK
    pltpu.sync_copy(dst_hbm.at[pl.ds(off, ECHUNK)], dst_v)
    pltpu.sync_copy(ones_v, acc.at[dst_v], add=True)
    return _
  lax.fori_loop(0, N_ECH, deg_chunk, None)

  plsc.subcore_barrier()
  _wb_acc(c, s, acc, d0_hbm, d1_hbm)

  _zero_acc(s, acc, zbuf)
  plsc.subcore_barrier()

  def edge_chunk(j, _):
    off = base + j * ECHUNK
    pltpu.sync_copy(src_hbm.at[pl.ds(off, ECHUNK)], idx_v)
    pltpu.sync_copy(dst_hbm.at[pl.ds(off, ECHUNK)], dst_v)
    pltpu.async_copy(tab_hbm.at[idx_v], rows_v, sem).wait()
    pltpu.sync_copy(rows_v, acc.at[dst_v], add=True)
    return _
  lax.fori_loop(0, N_ECH, edge_chunk, None)

  plsc.subcore_barrier()
  _wb_acc(c, s, acc, p0_hbm, p1_hbm)


def _hop2_body(src_hbm, dst_hbm, tab_hbm, p0_hbm, p1_hbm,
               acc, idx_v, dst_v, rows_v, zbuf, sem):
  c = lax.axis_index("c")
  s = lax.axis_index("s")
  wid = s * NC + c
  base = wid * E_PER_W

  _fill(zbuf, RBLK, 0.0)
  _zero_acc(s, acc, zbuf)
  plsc.subcore_barrier()

  def edge_chunk(j, _):
    off = base + j * ECHUNK
    pltpu.sync_copy(src_hbm.at[pl.ds(off, ECHUNK)], idx_v)
    pltpu.sync_copy(dst_hbm.at[pl.ds(off, ECHUNK)], dst_v)
    pltpu.async_copy(tab_hbm.at[idx_v], rows_v, sem).wait()
    pltpu.sync_copy(rows_v, acc.at[dst_v], add=True)
    return _
  lax.fori_loop(0, N_ECH, edge_chunk, None)

  plsc.subcore_barrier()
  _wb_acc(c, s, acc, p0_hbm, p1_hbm)


_nd = jax.ShapeDtypeStruct((N_NODES, D), _f32)

_hop1 = pl.kernel(
    _hop1_body,
    out_type=(_nd, _nd, _nd, _nd),
    mesh=_mesh,
    scratch_types=(pltpu.VMEM_SHARED((N_NODES, D), _f32),
                   pltpu.VMEM((ECHUNK,), jnp.int32),
                   pltpu.VMEM((ECHUNK,), jnp.int32),
                   pltpu.VMEM((ECHUNK, D), _f32),
                   pltpu.VMEM((ECHUNK, D), _f32),
                   pltpu.VMEM((RBLK, D), _f32),
                   pltpu.SemaphoreType.DMA))

_hop2 = pl.kernel(
    _hop2_body,
    out_type=(_nd, _nd),
    mesh=_mesh,
    scratch_types=(pltpu.VMEM_SHARED((N_NODES, D), _f32),
                   pltpu.VMEM((ECHUNK,), jnp.int32),
                   pltpu.VMEM((ECHUNK,), jnp.int32),
                   pltpu.VMEM((ECHUNK, D), _f32),
                   pltpu.VMEM((RBLK, D), _f32),
                   pltpu.SemaphoreType.DMA))


def _combine_body(p0, p1, d0h, d1h, x1_hbm, rdeg_hbm,
                  b0, b1, db0, db1, rbuf):
  c = lax.axis_index("c")
  s = lax.axis_index("s")
  wid = s * NC + c

  def blk_body(k, _):
    blk = wid + NW * k
    @pl.when(blk < N_RBLK)
    def _():
      sl = pl.ds(blk * RBLK, RBLK)
      pltpu.sync_copy(p0.at[sl], b0)
      pltpu.sync_copy(p1.at[sl], b1)
      pltpu.sync_copy(d0h.at[sl], db0)
      pltpu.sync_copy(d1h.at[sl], db1)
      for r in range(RBLK):
        degv = db0[r, pl.ds(0, L)] + db1[r, pl.ds(0, L)]
        rv = 1.0 / jnp.maximum(degv, 1.0)
        rs = rv[0]
        for q in range(D // L):
          qsl = pl.ds(q * L, L)
          rbuf[r, qsl] = jnp.broadcast_to(rs, (L,))
          b0[r, qsl] = (b0[r, qsl] + b1[r, qsl]) * rs
      pltpu.sync_copy(b0, x1_hbm.at[sl])
      pltpu.sync_copy(rbuf, rdeg_hbm.at[sl])
    return _
  lax.fori_loop(0, (N_RBLK + NW - 1) // NW, blk_body, None)


_combine = pl.kernel(
    _combine_body,
    out_type=(_nd, _nd),
    mesh=_mesh,
    scratch_types=(pltpu.VMEM((RBLK, D), _f32),
                   pltpu.VMEM((RBLK, D), _f32),
                   pltpu.VMEM((RBLK, D), _f32),
                   pltpu.VMEM((RBLK, D), _f32),
                   pltpu.VMEM((RBLK, D), _f32)))


def _final_body(q0, q1, rdeg_hbm, tab_hbm, ids_hbm, f0_hbm, f1_hbm,
                idx_v, b0, b1, dd, fb, sem):
  c = lax.axis_index("c")
  s = lax.axis_index("s")
  wid = s * NC + c
  base = wid * B_PER_W

  pltpu.sync_copy(ids_hbm.at[pl.ds(base, B_PER_W)], idx_v)
  pltpu.async_copy(q0.at[idx_v], b0, sem).wait()
  pltpu.async_copy(q1.at[idx_v], b1, sem).wait()
  pltpu.async_copy(rdeg_hbm.at[idx_v], dd, sem).wait()
  pltpu.async_copy(tab_hbm.at[idx_v], fb, sem).wait()

  def row_body(r, _):
    rs = dd[r, pl.ds(0, L)][0]
    for q in range(D // L):
      qsl = pl.ds(q * L, L)
      b0[r, qsl] = (b0[r, qsl] + b1[r, qsl]) * rs
    return _
  lax.fori_loop(0, B_PER_W, row_body, None)

  pltpu.sync_copy(b0, f0_hbm.at[pl.ds(base, B_PER_W)])
  pltpu.sync_copy(fb, f1_hbm.at[pl.ds(base, B_PER_W)])


_final = pl.kernel(
    _final_body,
    out_type=(jax.ShapeDtypeStruct((B, D), _f32),
              jax.ShapeDtypeStruct((B, D), _f32)),
    mesh=_mesh,
    scratch_types=(pltpu.VMEM((B_PER_W,), jnp.int32),
                   pltpu.VMEM((B_PER_W, D), _f32),
                   pltpu.VMEM((B_PER_W, D), _f32),
                   pltpu.VMEM((B_PER_W, D), _f32),
                   pltpu.VMEM((B_PER_W, D), _f32),
                   pltpu.SemaphoreType.DMA))


TC_BLK = 512


def _tc_body(f0_ref, f1_ref, w_ref, b_ref, o_ref):
  acc = jnp.dot(f0_ref[...], w_ref[...], preferred_element_type=_f32)
  sc = jnp.dot(f1_ref[...], b_ref[...], preferred_element_type=_f32)
  o_ref[...] = jnp.maximum(acc + sc, 0.0)


_tc_epilogue = pl.pallas_call(
    _tc_body,
    grid=(B // TC_BLK,),
    in_specs=[
        pl.BlockSpec((TC_BLK, D), lambda i: (i, 0)),
        pl.BlockSpec((TC_BLK, D), lambda i: (i, 0)),
        pl.BlockSpec((D, D), lambda i: (0, 0)),
        pl.BlockSpec((D, 1), lambda i: (0, 0)),
    ],
    out_specs=pl.BlockSpec((TC_BLK, D), lambda i: (i, 0)),
    out_shape=jax.ShapeDtypeStruct((B, D), _f32),
)


def kernel(inputs, edge_index, table, weight, bias):
  ids = inputs.astype(jnp.int32)
  src = edge_index[0].astype(jnp.int32)
  dst = edge_index[1].astype(jnp.int32)
  table = table.astype(_f32)

  p0, p1, d0, d1 = _hop1(src, dst, table)
  x1, rdeg = _combine(p0, p1, d0, d1)
  q0, q1 = _hop2(src, dst, x1)
  f0, f1 = _final(q0, q1, rdeg, table, ids)
  return _tc_epilogue(f0, f1, weight, bias.reshape(D, 1))


# speedup vs baseline: 6.0721x; 1.5635x over previous
import jax
import jax.numpy as jnp
from jax import lax
from jax.experimental import pallas as pl
from jax.experimental.pallas import tpu as pltpu
from jax.experimental.pallas import tpu_sc as plsc

NC, NS, L = 2, 16, 16
NW = NC * NS
N_NODES = 10000
D = 128
N_EDGES = 320000
B = 4096

E_PER_W = N_EDGES // NW
ECHUNK = 40
N_ECH = E_PER_W // ECHUNK
RBLK = 16
N_RBLK = N_NODES // RBLK
B_PER_W = B // NW

_mesh = plsc.VectorSubcoreMesh(
    core_axis_name="c", subcore_axis_name="s", num_cores=NC, num_subcores=NS)

_f32 = jnp.float32


def _fill(buf, rows, value):
  v = jnp.full((L,), value, _f32)
  for r in range(rows):
    for q in range(D // L):
      buf[r, pl.ds(q * L, L)] = v


def _sliced_loop(s, body):
  def step(k, _):
    blk = s + NS * k
    @pl.when(blk < N_RBLK)
    def _():
      body(blk)
    return _
  lax.fori_loop(0, (N_RBLK + NS - 1) // NS, step, None)


def _zero_acc(s, acc, zbuf):
  _sliced_loop(s, lambda blk: pltpu.sync_copy(
      zbuf, acc.at[pl.ds(blk * RBLK, RBLK)]))


def _wb_acc(c, s, acc, hbm0, hbm1):
  def wb(blk):
    sl = pl.ds(blk * RBLK, RBLK)
    @pl.when(c == 0)
    def _():
      pltpu.sync_copy(acc.at[sl], hbm0.at[sl])
    @pl.when(c == 1)
    def _():
      pltpu.sync_copy(acc.at[sl], hbm1.at[sl])
  _sliced_loop(s, wb)


GDEPTH = 5
N_GRP = N_ECH // GDEPTH


def _deg_pass(base, dst_hbm, acc, dst_vs, ones_v, psems, ssems):
  def group(g, _):
    j0 = g * GDEPTH
    d_descs = []
    for b in range(GDEPTH):
      off = base + (j0 + b) * ECHUNK
      d_descs.append(pltpu.async_copy(
          dst_hbm.at[pl.ds(off, ECHUNK)], dst_vs[b], psems[b]))
    s_descs = []
    for b in range(GDEPTH):
      d_descs[b].wait()
      s_descs.append(pltpu.async_copy(
          ones_v, acc.at[dst_vs[b]], ssems[b], add=True))
    for b in range(GDEPTH):
      s_descs[b].wait()
    return _
  lax.fori_loop(0, N_GRP, group, None)


def _rows_pass(base, src_hbm, dst_hbm, tab_hbm, acc,
               idx_vs, dst_vs, rows_vs, psems, gsems, ssems):
  def group(g, _):
    j0 = g * GDEPTH
    i_descs = []
    for b in range(GDEPTH):
      off = base + (j0 + b) * ECHUNK
      i_descs.append((
          pltpu.async_copy(src_hbm.at[pl.ds(off, ECHUNK)], idx_vs[b], psems[b]),
          pltpu.async_copy(dst_hbm.at[pl.ds(off, ECHUNK)], dst_vs[b], psems[b])))
    g_descs = []
    for b in range(GDEPTH):
      i_descs[b][0].wait()
      g_descs.append(pltpu.async_copy(tab_hbm.at[idx_vs[b]], rows_vs[b], gsems[b]))
    s_descs = []
    for b in range(GDEPTH):
      g_descs[b].wait()
      i_descs[b][1].wait()
      s_descs.append(pltpu.async_copy(
          rows_vs[b], acc.at[dst_vs[b]], ssems[b], add=True))
    for b in range(GDEPTH):
      s_descs[b].wait()
    return _
  lax.fori_loop(0, N_GRP, group, None)


def _hop1_body(src_hbm, dst_hbm, tab_hbm,
               p0_hbm, p1_hbm, d0_hbm, d1_hbm, acc, *rest):
  idx_vs = rest[0:GDEPTH]
  dst_vs = rest[GDEPTH:2 * GDEPTH]
  rows_vs = rest[2 * GDEPTH:3 * GDEPTH]
  ones_v, zbuf = rest[3 * GDEPTH:3 * GDEPTH + 2]
  sems = rest[3 * GDEPTH + 2:]
  psems, gsems, ssems = (sems[0:GDEPTH], sems[GDEPTH:2 * GDEPTH],
                         sems[2 * GDEPTH:3 * GDEPTH])
  c = lax.axis_index("c")
  s = lax.axis_index("s")
  wid = s * NC + c
  base = wid * E_PER_W

  _fill(zbuf, RBLK, 0.0)
  _fill(ones_v, ECHUNK, 1.0)

  _zero_acc(s, acc, zbuf)
  plsc.subcore_barrier()
  _deg_pass(base, dst_hbm, acc, dst_vs, ones_v, psems, ssems)
  plsc.subcore_barrier()
  _wb_acc(c, s, acc, d0_hbm, d1_hbm)

  _zero_acc(s, acc, zbuf)
  plsc.subcore_barrier()
  _rows_pass(base, src_hbm, dst_hbm, tab_hbm, acc,
             idx_vs, dst_vs, rows_vs, psems, gsems, ssems)
  plsc.subcore_barrier()
  _wb_acc(c, s, acc, p0_hbm, p1_hbm)


def _hop2_body(src_hbm, dst_hbm, tab_hbm, p0_hbm, p1_hbm, acc, *rest):
  idx_vs = rest[0:GDEPTH]
  dst_vs = rest[GDEPTH:2 * GDEPTH]
  rows_vs = rest[2 * GDEPTH:3 * GDEPTH]
  zbuf = rest[3 * GDEPTH]
  sems = rest[3 * GDEPTH + 1:]
  psems, gsems, ssems = (sems[0:GDEPTH], sems[GDEPTH:2 * GDEPTH],
                         sems[2 * GDEPTH:3 * GDEPTH])
  c = lax.axis_index("c")
  s = lax.axis_index("s")
  wid = s * NC + c
  base = wid * E_PER_W

  _fill(zbuf, RBLK, 0.0)
  _zero_acc(s, acc, zbuf)
  plsc.subcore_barrier()
  _rows_pass(base, src_hbm, dst_hbm, tab_hbm, acc,
             idx_vs, dst_vs, rows_vs, psems, gsems, ssems)
  plsc.subcore_barrier()
  _wb_acc(c, s, acc, p0_hbm, p1_hbm)


_nd = jax.ShapeDtypeStruct((N_NODES, D), _f32)

_hop1 = pl.kernel(
    _hop1_body,
    out_type=(_nd, _nd, _nd, _nd),
    mesh=_mesh,
    scratch_types=(
        (pltpu.VMEM_SHARED((N_NODES, D), _f32),)
        + tuple(pltpu.VMEM((ECHUNK,), jnp.int32) for _ in range(GDEPTH))
        + tuple(pltpu.VMEM((ECHUNK,), jnp.int32) for _ in range(GDEPTH))
        + tuple(pltpu.VMEM((ECHUNK, D), _f32) for _ in range(GDEPTH))
        + (pltpu.VMEM((ECHUNK, D), _f32), pltpu.VMEM((RBLK, D), _f32))
        + tuple(pltpu.SemaphoreType.DMA for _ in range(3 * GDEPTH))))

_hop2 = pl.kernel(
    _hop2_body,
    out_type=(_nd, _nd),
    mesh=_mesh,
    scratch_types=(
        (pltpu.VMEM_SHARED((N_NODES, D), _f32),)
        + tuple(pltpu.VMEM((ECHUNK,), jnp.int32) for _ in range(GDEPTH))
        + tuple(pltpu.VMEM((ECHUNK,), jnp.int32) for _ in range(GDEPTH))
        + tuple(pltpu.VMEM((ECHUNK, D), _f32) for _ in range(GDEPTH))
        + (pltpu.VMEM((RBLK, D), _f32),)
        + tuple(pltpu.SemaphoreType.DMA for _ in range(3 * GDEPTH))))


def _combine_body(p0, p1, d0h, d1h, x1_hbm, rdeg_hbm,
                  b0, b1, db0, db1, rbuf):
  c = lax.axis_index("c")
  s = lax.axis_index("s")
  wid = s * NC + c

  def blk_body(k, _):
    blk = wid + NW * k
    @pl.when(blk < N_RBLK)
    def _():
      sl = pl.ds(blk * RBLK, RBLK)
      pltpu.sync_copy(p0.at[sl], b0)
      pltpu.sync_copy(p1.at[sl], b1)
      pltpu.sync_copy(d0h.at[sl], db0)
      pltpu.sync_copy(d1h.at[sl], db1)
      for r in range(RBLK):
        degv = db0[r, pl.ds(0, L)] + db1[r, pl.ds(0, L)]
        rv = 1.0 / jnp.maximum(degv, 1.0)
        rs = rv[0]
        for q in range(D // L):
          qsl = pl.ds(q * L, L)
          rbuf[r, qsl] = jnp.broadcast_to(rs, (L,))
          b0[r, qsl] = (b0[r, qsl] + b1[r, qsl]) * rs
      pltpu.sync_copy(b0, x1_hbm.at[sl])
      pltpu.sync_copy(rbuf, rdeg_hbm.at[sl])
    return _
  lax.fori_loop(0, (N_RBLK + NW - 1) // NW, blk_body, None)


_combine = pl.kernel(
    _combine_body,
    out_type=(_nd, _nd),
    mesh=_mesh,
    scratch_types=(pltpu.VMEM((RBLK, D), _f32),
                   pltpu.VMEM((RBLK, D), _f32),
                   pltpu.VMEM((RBLK, D), _f32),
                   pltpu.VMEM((RBLK, D), _f32),
                   pltpu.VMEM((RBLK, D), _f32)))


def _final_body(q0, q1, rdeg_hbm, tab_hbm, ids_hbm, f0_hbm, f1_hbm,
                idx_v, b0, b1, dd, fb, sem):
  c = lax.axis_index("c")
  s = lax.axis_index("s")
  wid = s * NC + c
  base = wid * B_PER_W

  pltpu.sync_copy(ids_hbm.at[pl.ds(base, B_PER_W)], idx_v)
  pltpu.async_copy(q0.at[idx_v], b0, sem).wait()
  pltpu.async_copy(q1.at[idx_v], b1, sem).wait()
  pltpu.async_copy(rdeg_hbm.at[idx_v], dd, sem).wait()
  pltpu.async_copy(tab_hbm.at[idx_v], fb, sem).wait()

  def row_body(r, _):
    rs = dd[r, pl.ds(0, L)][0]
    for q in range(D // L):
      qsl = pl.ds(q * L, L)
      b0[r, qsl] = (b0[r, qsl] + b1[r, qsl]) * rs
    return _
  lax.fori_loop(0, B_PER_W, row_body, None)

  pltpu.sync_copy(b0, f0_hbm.at[pl.ds(base, B_PER_W)])
  pltpu.sync_copy(fb, f1_hbm.at[pl.ds(base, B_PER_W)])


_final = pl.kernel(
    _final_body,
    out_type=(jax.ShapeDtypeStruct((B, D), _f32),
              jax.ShapeDtypeStruct((B, D), _f32)),
    mesh=_mesh,
    scratch_types=(pltpu.VMEM((B_PER_W,), jnp.int32),
                   pltpu.VMEM((B_PER_W, D), _f32),
                   pltpu.VMEM((B_PER_W, D), _f32),
                   pltpu.VMEM((B_PER_W, D), _f32),
                   pltpu.VMEM((B_PER_W, D), _f32),
                   pltpu.SemaphoreType.DMA))


TC_BLK = 512


def _tc_body(f0_ref, f1_ref, w_ref, b_ref, o_ref):
  acc = jnp.dot(f0_ref[...], w_ref[...], preferred_element_type=_f32)
  sc = jnp.dot(f1_ref[...], b_ref[...], preferred_element_type=_f32)
  o_ref[...] = jnp.maximum(acc + sc, 0.0)


_tc_epilogue = pl.pallas_call(
    _tc_body,
    grid=(B // TC_BLK,),
    in_specs=[
        pl.BlockSpec((TC_BLK, D), lambda i: (i, 0)),
        pl.BlockSpec((TC_BLK, D), lambda i: (i, 0)),
        pl.BlockSpec((D, D), lambda i: (0, 0)),
        pl.BlockSpec((D, 1), lambda i: (0, 0)),
    ],
    out_specs=pl.BlockSpec((TC_BLK, D), lambda i: (i, 0)),
    out_shape=jax.ShapeDtypeStruct((B, D), _f32),
)


def kernel(inputs, edge_index, table, weight, bias):
  ids = inputs.astype(jnp.int32)
  src = edge_index[0].astype(jnp.int32)
  dst = edge_index[1].astype(jnp.int32)
  table = table.astype(_f32)

  p0, p1, d0, d1 = _hop1(src, dst, table)
  x1, rdeg = _combine(p0, p1, d0, d1)
  q0, q1 = _hop2(src, dst, x1)
  f0, f1 = _final(q0, q1, rdeg, table, ids)
  return _tc_epilogue(f0, f1, weight, bias.reshape(D, 1))


# speedup vs baseline: 6.5607x; 1.0805x over previous
import jax
import jax.numpy as jnp
from jax import lax
from jax.experimental import pallas as pl
from jax.experimental.pallas import tpu as pltpu
from jax.experimental.pallas import tpu_sc as plsc

NC, NS, L = 2, 16, 16
NW = NC * NS
N_NODES = 10000
D = 128
N_EDGES = 320000
B = 4096

E_PER_W = N_EDGES // NW
ECHUNK = 40
N_ECH = E_PER_W // ECHUNK
RBLK = 16
N_RBLK = N_NODES // RBLK
B_PER_W = B // NW

_mesh = plsc.VectorSubcoreMesh(
    core_axis_name="c", subcore_axis_name="s", num_cores=NC, num_subcores=NS)

_f32 = jnp.float32


def _fill(buf, rows, value):
  v = jnp.full((L,), value, _f32)
  for r in range(rows):
    for q in range(D // L):
      buf[r, pl.ds(q * L, L)] = v


def _sliced_loop(s, body):
  def step(k, _):
    blk = s + NS * k
    @pl.when(blk < N_RBLK)
    def _():
      body(blk)
    return _
  lax.fori_loop(0, (N_RBLK + NS - 1) // NS, step, None)


def _zero_acc(s, acc, zbuf):
  _sliced_loop(s, lambda blk: pltpu.sync_copy(
      zbuf, acc.at[pl.ds(blk * RBLK, RBLK)]))


def _wb_acc(c, s, acc, hbm0, hbm1):
  def wb(blk):
    sl = pl.ds(blk * RBLK, RBLK)
    @pl.when(c == 0)
    def _():
      pltpu.sync_copy(acc.at[sl], hbm0.at[sl])
    @pl.when(c == 1)
    def _():
      pltpu.sync_copy(acc.at[sl], hbm1.at[sl])
  _sliced_loop(s, wb)


GDEPTH = 5
N_GRP = N_ECH // GDEPTH


def _deg_pass(base, dst_hbm, acc, dst_vs, ones_v, psems, ssems):
  def group(g, _):
    j0 = g * GDEPTH
    d_descs = []
    for b in range(GDEPTH):
      off = base + (j0 + b) * ECHUNK
      d_descs.append(pltpu.async_copy(
          dst_hbm.at[pl.ds(off, ECHUNK)], dst_vs[b], psems[b]))
    s_descs = []
    for b in range(GDEPTH):
      d_descs[b].wait()
      s_descs.append(pltpu.async_copy(
          ones_v, acc.at[dst_vs[b]], ssems[b], add=True))
    for b in range(GDEPTH):
      s_descs[b].wait()
    return _
  lax.fori_loop(0, N_GRP, group, None)


def _rows_pass(base, src_hbm, dst_hbm, tab_hbm, acc,
               idx_vs, dst_vs, rows_vs, psems, gsems, ssems):
  def group(g, _):
    j0 = g * GDEPTH
    i_descs = []
    for b in range(GDEPTH):
      off = base + (j0 + b) * ECHUNK
      i_descs.append((
          pltpu.async_copy(src_hbm.at[pl.ds(off, ECHUNK)], idx_vs[b], psems[b]),
          pltpu.async_copy(dst_hbm.at[pl.ds(off, ECHUNK)], dst_vs[b], psems[b])))
    g_descs = []
    for b in range(GDEPTH):
      i_descs[b][0].wait()
      g_descs.append(pltpu.async_copy(tab_hbm.at[idx_vs[b]], rows_vs[b], gsems[b]))
    s_descs = []
    for b in range(GDEPTH):
      g_descs[b].wait()
      i_descs[b][1].wait()
      s_descs.append(pltpu.async_copy(
          rows_vs[b], acc.at[dst_vs[b]], ssems[b], add=True))
    for b in range(GDEPTH):
      s_descs[b].wait()
    return _
  lax.fori_loop(0, N_GRP, group, None)


def _hop1_body(src_hbm, dst_hbm, tab_hbm,
               p0_hbm, p1_hbm, d0_hbm, d1_hbm, acc, *rest):
  idx_vs = rest[0:GDEPTH]
  dst_vs = rest[GDEPTH:2 * GDEPTH]
  rows_vs = rest[2 * GDEPTH:3 * GDEPTH]
  ones_v, zbuf = rest[3 * GDEPTH:3 * GDEPTH + 2]
  sems = rest[3 * GDEPTH + 2:]
  psems, gsems, ssems = (sems[0:GDEPTH], sems[GDEPTH:2 * GDEPTH],
                         sems[2 * GDEPTH:3 * GDEPTH])
  c = lax.axis_index("c")
  s = lax.axis_index("s")
  wid = s * NC + c
  base = wid * E_PER_W

  _fill(zbuf, RBLK, 0.0)
  _fill(ones_v, ECHUNK, 1.0)

  _zero_acc(s, acc, zbuf)
  plsc.subcore_barrier()
  _deg_pass(base, dst_hbm, acc, dst_vs, ones_v, psems, ssems)
  plsc.subcore_barrier()
  _wb_acc(c, s, acc, d0_hbm, d1_hbm)

  _zero_acc(s, acc, zbuf)
  plsc.subcore_barrier()
  _rows_pass(base, src_hbm, dst_hbm, tab_hbm, acc,
             idx_vs, dst_vs, rows_vs, psems, gsems, ssems)
  plsc.subcore_barrier()
  _wb_acc(c, s, acc, p0_hbm, p1_hbm)


def _hop2_body(src_hbm, dst_hbm, tab_hbm, p0_hbm, p1_hbm, acc, *rest):
  idx_vs = rest[0:GDEPTH]
  dst_vs = rest[GDEPTH:2 * GDEPTH]
  rows_vs = rest[2 * GDEPTH:3 * GDEPTH]
  zbuf = rest[3 * GDEPTH]
  sems = rest[3 * GDEPTH + 1:]
  psems, gsems, ssems = (sems[0:GDEPTH], sems[GDEPTH:2 * GDEPTH],
                         sems[2 * GDEPTH:3 * GDEPTH])
  c = lax.axis_index("c")
  s = lax.axis_index("s")
  wid = s * NC + c
  base = wid * E_PER_W

  _fill(zbuf, RBLK, 0.0)
  _zero_acc(s, acc, zbuf)
  plsc.subcore_barrier()
  _rows_pass(base, src_hbm, dst_hbm, tab_hbm, acc,
             idx_vs, dst_vs, rows_vs, psems, gsems, ssems)
  plsc.subcore_barrier()
  _wb_acc(c, s, acc, p0_hbm, p1_hbm)


_nd = jax.ShapeDtypeStruct((N_NODES, D), _f32)

_hop1 = pl.kernel(
    _hop1_body,
    out_type=(_nd, _nd, _nd, _nd),
    mesh=_mesh,
    scratch_types=(
        (pltpu.VMEM_SHARED((N_NODES, D), _f32),)
        + tuple(pltpu.VMEM((ECHUNK,), jnp.int32) for _ in range(GDEPTH))
        + tuple(pltpu.VMEM((ECHUNK,), jnp.int32) for _ in range(GDEPTH))
        + tuple(pltpu.VMEM((ECHUNK, D), _f32) for _ in range(GDEPTH))
        + (pltpu.VMEM((ECHUNK, D), _f32), pltpu.VMEM((RBLK, D), _f32))
        + tuple(pltpu.SemaphoreType.DMA for _ in range(3 * GDEPTH))))

_hop2 = pl.kernel(
    _hop2_body,
    out_type=(_nd, _nd),
    mesh=_mesh,
    scratch_types=(
        (pltpu.VMEM_SHARED((N_NODES, D), _f32),)
        + tuple(pltpu.VMEM((ECHUNK,), jnp.int32) for _ in range(GDEPTH))
        + tuple(pltpu.VMEM((ECHUNK,), jnp.int32) for _ in range(GDEPTH))
        + tuple(pltpu.VMEM((ECHUNK, D), _f32) for _ in range(GDEPTH))
        + (pltpu.VMEM((RBLK, D), _f32),)
        + tuple(pltpu.SemaphoreType.DMA for _ in range(3 * GDEPTH))))


def _combine_body(p0, p1, d0h, d1h, x1_hbm, rdeg_hbm, *rest):
  bufs = [rest[0:5], rest[5:10]]
  lsems = [rest[10:14], rest[14:18]]
  wsems = [rest[18:20], rest[20:22]]
  c = lax.axis_index("c")
  s = lax.axis_index("s")
  wid = s * NC + c

  def load(blk, bs, ls):
    sl = pl.ds(blk * RBLK, RBLK)
    return [pltpu.async_copy(src.at[sl], dst, sem)
            for src, dst, sem in zip((p0, p1, d0h, d1h), bs[:4], ls)]

  def compute_store(blk, bs, ws):
    b0, b1, db0, db1, rbuf = bs
    sl = pl.ds(blk * RBLK, RBLK)
    for r in range(RBLK):
      degv = db0[r, pl.ds(0, L)] + db1[r, pl.ds(0, L)]
      rv = 1.0 / jnp.maximum(degv, 1.0)
      rs = rv[0]
      for q in range(D // L):
        qsl = pl.ds(q * L, L)
        rbuf[r, qsl] = jnp.broadcast_to(rs, (L,))
        b0[r, qsl] = (b0[r, qsl] + b1[r, qsl]) * rs
    return [pltpu.async_copy(b0, x1_hbm.at[sl], ws[0]),
            pltpu.async_copy(rbuf, rdeg_hbm.at[sl], ws[1])]

  def pair(k2, _):
    blk_a = wid + NW * (2 * k2)
    blk_b = wid + NW * (2 * k2 + 1)

    @pl.when(blk_a < N_RBLK)
    def _():
      la = load(blk_a, bufs[0], lsems[0])

      @pl.when(blk_b < N_RBLK)
      def _():
        lb = load(blk_b, bufs[1], lsems[1])
        for d in la:
          d.wait()
        wa = compute_store(blk_a, bufs[0], wsems[0])
        for d in lb:
          d.wait()
        wb = compute_store(blk_b, bufs[1], wsems[1])
        for d in wa + wb:
          d.wait()

      @pl.when(blk_b >= N_RBLK)
      def _():
        for d in la:
          d.wait()
        wa = compute_store(blk_a, bufs[0], wsems[0])
        for d in wa:
          d.wait()
    return _
  lax.fori_loop(0, (N_RBLK + 2 * NW - 1) // (2 * NW), pair, None)


_combine = pl.kernel(
    _combine_body,
    out_type=(_nd, _nd),
    mesh=_mesh,
    scratch_types=(
        tuple(pltpu.VMEM((RBLK, D), _f32) for _ in range(10))
        + tuple(pltpu.SemaphoreType.DMA for _ in range(12))))


def _final_body(q0, q1, rdeg_hbm, tab_hbm, ids_hbm, f0_hbm, f1_hbm,
                idx_v, b0, b1, dd, fb, sem):
  c = lax.axis_index("c")
  s = lax.axis_index("s")
  wid = s * NC + c
  base = wid * B_PER_W

  pltpu.sync_copy(ids_hbm.at[pl.ds(base, B_PER_W)], idx_v)
  pltpu.async_copy(q0.at[idx_v], b0, sem).wait()
  pltpu.async_copy(q1.at[idx_v], b1, sem).wait()
  pltpu.async_copy(rdeg_hbm.at[idx_v], dd, sem).wait()
  pltpu.async_copy(tab_hbm.at[idx_v], fb, sem).wait()

  def row_body(r, _):
    rs = dd[r, pl.ds(0, L)][0]
    for q in range(D // L):
      qsl = pl.ds(q * L, L)
      b0[r, qsl] = (b0[r, qsl] + b1[r, qsl]) * rs
    return _
  lax.fori_loop(0, B_PER_W, row_body, None)

  pltpu.sync_copy(b0, f0_hbm.at[pl.ds(base, B_PER_W)])
  pltpu.sync_copy(fb, f1_hbm.at[pl.ds(base, B_PER_W)])


_final = pl.kernel(
    _final_body,
    out_type=(jax.ShapeDtypeStruct((B, D), _f32),
              jax.ShapeDtypeStruct((B, D), _f32)),
    mesh=_mesh,
    scratch_types=(pltpu.VMEM((B_PER_W,), jnp.int32),
                   pltpu.VMEM((B_PER_W, D), _f32),
                   pltpu.VMEM((B_PER_W, D), _f32),
                   pltpu.VMEM((B_PER_W, D), _f32),
                   pltpu.VMEM((B_PER_W, D), _f32),
                   pltpu.SemaphoreType.DMA))


TC_BLK = 512


def _tc_body(f0_ref, f1_ref, w_ref, b_ref, o_ref):
  acc = jnp.dot(f0_ref[...], w_ref[...], preferred_element_type=_f32)
  sc = jnp.dot(f1_ref[...], b_ref[...], preferred_element_type=_f32)
  o_ref[...] = jnp.maximum(acc + sc, 0.0)


_tc_epilogue = pl.pallas_call(
    _tc_body,
    grid=(B // TC_BLK,),
    in_specs=[
        pl.BlockSpec((TC_BLK, D), lambda i: (i, 0)),
        pl.BlockSpec((TC_BLK, D), lambda i: (i, 0)),
        pl.BlockSpec((D, D), lambda i: (0, 0)),
        pl.BlockSpec((D, 1), lambda i: (0, 0)),
    ],
    out_specs=pl.BlockSpec((TC_BLK, D), lambda i: (i, 0)),
    out_shape=jax.ShapeDtypeStruct((B, D), _f32),
)


def kernel(inputs, edge_index, table, weight, bias):
  ids = inputs.astype(jnp.int32)
  src = edge_index[0].astype(jnp.int32)
  dst = edge_index[1].astype(jnp.int32)
  table = table.astype(_f32)

  p0, p1, d0, d1 = _hop1(src, dst, table)
  x1, rdeg = _combine(p0, p1, d0, d1)
  q0, q1 = _hop2(src, dst, x1)
  f0, f1 = _final(q0, q1, rdeg, table, ids)
  return _tc_epilogue(f0, f1, weight, bias.reshape(D, 1))


# speedup vs baseline: 7.2281x; 1.1017x over previous
import jax
import jax.numpy as jnp
from jax import lax
from jax.experimental import pallas as pl
from jax.experimental.pallas import tpu as pltpu
from jax.experimental.pallas import tpu_sc as plsc

NC, NS, L = 2, 16, 16
NW = NC * NS
N_NODES = 10000
D = 128
N_EDGES = 320000
B = 4096

E_PER_W = N_EDGES // NW
ECHUNK = 40
N_ECH = E_PER_W // ECHUNK
RBLK = 16
N_RBLK = N_NODES // RBLK
B_PER_W = B // NW

_mesh = plsc.VectorSubcoreMesh(
    core_axis_name="c", subcore_axis_name="s", num_cores=NC, num_subcores=NS)

_f32 = jnp.float32


def _fill(buf, rows, value):
  v = jnp.full((L,), value, _f32)
  for r in range(rows):
    for q in range(D // L):
      buf[r, pl.ds(q * L, L)] = v


def _sliced_loop(s, body):
  def step(k, _):
    blk = s + NS * k
    @pl.when(blk < N_RBLK)
    def _():
      body(blk)
    return _
  lax.fori_loop(0, (N_RBLK + NS - 1) // NS, step, None)


def _zero_acc(s, acc, zbuf):
  _sliced_loop(s, lambda blk: pltpu.sync_copy(
      zbuf, acc.at[pl.ds(blk * RBLK, RBLK)]))


def _wb_acc(c, s, acc, hbm0, hbm1):
  def wb(blk):
    sl = pl.ds(blk * RBLK, RBLK)
    @pl.when(c == 0)
    def _():
      pltpu.sync_copy(acc.at[sl], hbm0.at[sl])
    @pl.when(c == 1)
    def _():
      pltpu.sync_copy(acc.at[sl], hbm1.at[sl])
  _sliced_loop(s, wb)


GDEPTH = 5
N_GRP = N_ECH // GDEPTH


def _deg_pass(base, dst_hbm, acc, dst_vs, ones_v, psems, ssems):
  def group(g, _):
    j0 = g * GDEPTH
    d_descs = []
    for b in range(GDEPTH):
      off = base + (j0 + b) * ECHUNK
      d_descs.append(pltpu.async_copy(
          dst_hbm.at[pl.ds(off, ECHUNK)], dst_vs[b], psems[b]))
    s_descs = []
    for b in range(GDEPTH):
      d_descs[b].wait()
      s_descs.append(pltpu.async_copy(
          ones_v, acc.at[dst_vs[b]], ssems[b], add=True))
    for b in range(GDEPTH):
      s_descs[b].wait()
    return _
  lax.fori_loop(0, N_GRP, group, None)


def _rows_pass(base, src_hbm, dst_hbm, tab_hbm, acc,
               idx_vs, dst_vs, rows_vs, psems, gsems, ssems):
  def idx_load(j, b):
    off = jnp.minimum(base + j * ECHUNK, N_EDGES - ECHUNK)
    return (pltpu.async_copy(src_hbm.at[pl.ds(off, ECHUNK)], idx_vs[b], psems[b]),
            pltpu.async_copy(dst_hbm.at[pl.ds(off, ECHUNK)], dst_vs[b], psems[b]))

  def gather(b):
    return pltpu.async_copy(tab_hbm.at[idx_vs[b]], rows_vs[b], gsems[b])

  def scatter(b):
    return pltpu.async_copy(rows_vs[b], acc.at[dst_vs[b]], ssems[b], add=True)

  p_descs = [idx_load(b, b) for b in range(GDEPTH)]
  g_descs = []
  for b in range(GDEPTH):
    p_descs[b][0].wait()
    p_descs[b][1].wait()
    g_descs.append(gather(b))

  def group(g, _):
    s_descs = []
    for b in range(GDEPTH):
      pltpu.make_async_copy(tab_hbm.at[idx_vs[b]], rows_vs[b], gsems[b]).wait()
      s_descs.append(scatter(b))
    i_next = []
    for b in range(GDEPTH):
      s_descs[b].wait()
      i_next.append(idx_load((g + 1) * GDEPTH + b, b))
    for b in range(GDEPTH):
      i_next[b][0].wait()
      i_next[b][1].wait()
      gather(b)
    return _
  lax.fori_loop(0, N_GRP - 1, group, None)

  s_descs = []
  for b in range(GDEPTH):
    pltpu.make_async_copy(tab_hbm.at[idx_vs[b]], rows_vs[b], gsems[b]).wait()
    s_descs.append(scatter(b))
  for b in range(GDEPTH):
    s_descs[b].wait()


def _hop1_body(src_hbm, dst_hbm, tab_hbm,
               p0_hbm, p1_hbm, d0_hbm, d1_hbm, acc, *rest):
  idx_vs = rest[0:GDEPTH]
  dst_vs = rest[GDEPTH:2 * GDEPTH]
  rows_vs = rest[2 * GDEPTH:3 * GDEPTH]
  ones_v, zbuf = rest[3 * GDEPTH:3 * GDEPTH + 2]
  sems = rest[3 * GDEPTH + 2:]
  psems, gsems, ssems = (sems[0:GDEPTH], sems[GDEPTH:2 * GDEPTH],
                         sems[2 * GDEPTH:3 * GDEPTH])
  c = lax.axis_index("c")
  s = lax.axis_index("s")
  wid = s * NC + c
  base = wid * E_PER_W

  _fill(zbuf, RBLK, 0.0)
  _fill(ones_v, ECHUNK, 1.0)

  _zero_acc(s, acc, zbuf)
  plsc.subcore_barrier()
  _deg_pass(base, dst_hbm, acc, dst_vs, ones_v, psems, ssems)
  plsc.subcore_barrier()
  _wb_acc(c, s, acc, d0_hbm, d1_hbm)

  _zero_acc(s, acc, zbuf)
  plsc.subcore_barrier()
  _rows_pass(base, src_hbm, dst_hbm, tab_hbm, acc,
             idx_vs, dst_vs, rows_vs, psems, gsems, ssems)
  plsc.subcore_barrier()
  _wb_acc(c, s, acc, p0_hbm, p1_hbm)


def _hop2_body(src_hbm, dst_hbm, tab_hbm, p0_hbm, p1_hbm, acc, *rest):
  idx_vs = rest[0:GDEPTH]
  dst_vs = rest[GDEPTH:2 * GDEPTH]
  rows_vs = rest[2 * GDEPTH:3 * GDEPTH]
  zbuf = rest[3 * GDEPTH]
  sems = rest[3 * GDEPTH + 1:]
  psems, gsems, ssems = (sems[0:GDEPTH], sems[GDEPTH:2 * GDEPTH],
                         sems[2 * GDEPTH:3 * GDEPTH])
  c = lax.axis_index("c")
  s = lax.axis_index("s")
  wid = s * NC + c
  base = wid * E_PER_W

  _fill(zbuf, RBLK, 0.0)
  _zero_acc(s, acc, zbuf)
  plsc.subcore_barrier()
  _rows_pass(base, src_hbm, dst_hbm, tab_hbm, acc,
             idx_vs, dst_vs, rows_vs, psems, gsems, ssems)
  plsc.subcore_barrier()
  _wb_acc(c, s, acc, p0_hbm, p1_hbm)


_nd = jax.ShapeDtypeStruct((N_NODES, D), _f32)

_hop1 = pl.kernel(
    _hop1_body,
    out_type=(_nd, _nd, _nd, _nd),
    mesh=_mesh,
    scratch_types=(
        (pltpu.VMEM_SHARED((N_NODES, D), _f32),)
        + tuple(pltpu.VMEM((ECHUNK,), jnp.int32) for _ in range(GDEPTH))
        + tuple(pltpu.VMEM((ECHUNK,), jnp.int32) for _ in range(GDEPTH))
        + tuple(pltpu.VMEM((ECHUNK, D), _f32) for _ in range(GDEPTH))
        + (pltpu.VMEM((ECHUNK, D), _f32), pltpu.VMEM((RBLK, D), _f32))
        + tuple(pltpu.SemaphoreType.DMA for _ in range(3 * GDEPTH))))

_hop2 = pl.kernel(
    _hop2_body,
    out_type=(_nd, _nd),
    mesh=_mesh,
    scratch_types=(
        (pltpu.VMEM_SHARED((N_NODES, D), _f32),)
        + tuple(pltpu.VMEM((ECHUNK,), jnp.int32) for _ in range(GDEPTH))
        + tuple(pltpu.VMEM((ECHUNK,), jnp.int32) for _ in range(GDEPTH))
        + tuple(pltpu.VMEM((ECHUNK, D), _f32) for _ in range(GDEPTH))
        + (pltpu.VMEM((RBLK, D), _f32),)
        + tuple(pltpu.SemaphoreType.DMA for _ in range(3 * GDEPTH))))


def _combine_body(p0, p1, d0h, d1h, x1_hbm, rdeg_hbm, *rest):
  bufs = [rest[0:5], rest[5:10]]
  lsems = [rest[10:14], rest[14:18]]
  wsems = [rest[18:20], rest[20:22]]
  c = lax.axis_index("c")
  s = lax.axis_index("s")
  wid = s * NC + c

  def load(blk, bs, ls):
    sl = pl.ds(blk * RBLK, RBLK)
    return [pltpu.async_copy(src.at[sl], dst, sem)
            for src, dst, sem in zip((p0, p1, d0h, d1h), bs[:4], ls)]

  def compute_store(blk, bs, ws):
    b0, b1, db0, db1, rbuf = bs
    sl = pl.ds(blk * RBLK, RBLK)
    for r in range(RBLK):
      degv = db0[r, pl.ds(0, L)] + db1[r, pl.ds(0, L)]
      rv = 1.0 / jnp.maximum(degv, 1.0)
      rs = rv[0]
      for q in range(D // L):
        qsl = pl.ds(q * L, L)
        rbuf[r, qsl] = jnp.broadcast_to(rs, (L,))
        b0[r, qsl] = (b0[r, qsl] + b1[r, qsl]) * rs
    return [pltpu.async_copy(b0, x1_hbm.at[sl], ws[0]),
            pltpu.async_copy(rbuf, rdeg_hbm.at[sl], ws[1])]

  def pair(k2, _):
    blk_a = wid + NW * (2 * k2)
    blk_b = wid + NW * (2 * k2 + 1)

    @pl.when(blk_a < N_RBLK)
    def _():
      la = load(blk_a, bufs[0], lsems[0])

      @pl.when(blk_b < N_RBLK)
      def _():
        lb = load(blk_b, bufs[1], lsems[1])
        for d in la:
          d.wait()
        wa = compute_store(blk_a, bufs[0], wsems[0])
        for d in lb:
          d.wait()
        wb = compute_store(blk_b, bufs[1], wsems[1])
        for d in wa + wb:
          d.wait()

      @pl.when(blk_b >= N_RBLK)
      def _():
        for d in la:
          d.wait()
        wa = compute_store(blk_a, bufs[0], wsems[0])
        for d in wa:
          d.wait()
    return _
  lax.fori_loop(0, (N_RBLK + 2 * NW - 1) // (2 * NW), pair, None)


_combine = pl.kernel(
    _combine_body,
    out_type=(_nd, _nd),
    mesh=_mesh,
    scratch_types=(
        tuple(pltpu.VMEM((RBLK, D), _f32) for _ in range(10))
        + tuple(pltpu.SemaphoreType.DMA for _ in range(12))))


def _final_body(q0, q1, rdeg_hbm, tab_hbm, ids_hbm, f0_hbm, f1_hbm,
                idx_v, b0, b1, dd, fb, sem):
  c = lax.axis_index("c")
  s = lax.axis_index("s")
  wid = s * NC + c
  base = wid * B_PER_W

  pltpu.sync_copy(ids_hbm.at[pl.ds(base, B_PER_W)], idx_v)
  pltpu.async_copy(q0.at[idx_v], b0, sem).wait()
  pltpu.async_copy(q1.at[idx_v], b1, sem).wait()
  pltpu.async_copy(rdeg_hbm.at[idx_v], dd, sem).wait()
  pltpu.async_copy(tab_hbm.at[idx_v], fb, sem).wait()

  def row_body(r, _):
    rs = dd[r, pl.ds(0, L)][0]
    for q in range(D // L):
      qsl = pl.ds(q * L, L)
      b0[r, qsl] = (b0[r, qsl] + b1[r, qsl]) * rs
    return _
  lax.fori_loop(0, B_PER_W, row_body, None)

  pltpu.sync_copy(b0, f0_hbm.at[pl.ds(base, B_PER_W)])
  pltpu.sync_copy(fb, f1_hbm.at[pl.ds(base, B_PER_W)])


_final = pl.kernel(
    _final_body,
    out_type=(jax.ShapeDtypeStruct((B, D), _f32),
              jax.ShapeDtypeStruct((B, D), _f32)),
    mesh=_mesh,
    scratch_types=(pltpu.VMEM((B_PER_W,), jnp.int32),
                   pltpu.VMEM((B_PER_W, D), _f32),
                   pltpu.VMEM((B_PER_W, D), _f32),
                   pltpu.VMEM((B_PER_W, D), _f32),
                   pltpu.VMEM((B_PER_W, D), _f32),
                   pltpu.SemaphoreType.DMA))


TC_BLK = 512


def _tc_body(f0_ref, f1_ref, w_ref, b_ref, o_ref):
  acc = jnp.dot(f0_ref[...], w_ref[...], preferred_element_type=_f32)
  sc = jnp.dot(f1_ref[...], b_ref[...], preferred_element_type=_f32)
  o_ref[...] = jnp.maximum(acc + sc, 0.0)


_tc_epilogue = pl.pallas_call(
    _tc_body,
    grid=(B // TC_BLK,),
    in_specs=[
        pl.BlockSpec((TC_BLK, D), lambda i: (i, 0)),
        pl.BlockSpec((TC_BLK, D), lambda i: (i, 0)),
        pl.BlockSpec((D, D), lambda i: (0, 0)),
        pl.BlockSpec((D, 1), lambda i: (0, 0)),
    ],
    out_specs=pl.BlockSpec((TC_BLK, D), lambda i: (i, 0)),
    out_shape=jax.ShapeDtypeStruct((B, D), _f32),
)


def kernel(inputs, edge_index, table, weight, bias):
  ids = inputs.astype(jnp.int32)
  src = edge_index[0].astype(jnp.int32)
  dst = edge_index[1].astype(jnp.int32)
  table = table.astype(_f32)

  p0, p1, d0, d1 = _hop1(src, dst, table)
  x1, rdeg = _combine(p0, p1, d0, d1)
  q0, q1 = _hop2(src, dst, x1)
  f0, f1 = _final(q0, q1, rdeg, table, ids)
  return _tc_epilogue(f0, f1, weight, bias.reshape(D, 1))


# speedup vs baseline: 7.4593x; 1.0320x over previous
import jax
import jax.numpy as jnp
from jax import lax
from jax.experimental import pallas as pl
from jax.experimental.pallas import tpu as pltpu
from jax.experimental.pallas import tpu_sc as plsc

NC, NS, L = 2, 16, 16
NW = NC * NS
N_NODES = 10000
D = 128
N_EDGES = 320000
B = 4096

E_PER_W = N_EDGES // NW
ECHUNK = 40
N_ECH = E_PER_W // ECHUNK
RBLK = 16
N_RBLK = N_NODES // RBLK
B_PER_W = B // NW

_mesh = plsc.VectorSubcoreMesh(
    core_axis_name="c", subcore_axis_name="s", num_cores=NC, num_subcores=NS)

_f32 = jnp.float32


def _fill(buf, rows, value):
  v = jnp.full((L,), value, _f32)
  for r in range(rows):
    for q in range(D // L):
      buf[r, pl.ds(q * L, L)] = v


def _sliced_loop(s, body):
  def step(k, _):
    blk = s + NS * k
    @pl.when(blk < N_RBLK)
    def _():
      body(blk)
    return _
  lax.fori_loop(0, (N_RBLK + NS - 1) // NS, step, None)


def _zero_acc(s, acc, zbuf):
  _sliced_loop(s, lambda blk: pltpu.sync_copy(
      zbuf, acc.at[pl.ds(blk * RBLK, RBLK)]))


def _wb_acc(c, s, acc, hbm0, hbm1):
  def wb(blk):
    sl = pl.ds(blk * RBLK, RBLK)
    @pl.when(c == 0)
    def _():
      pltpu.sync_copy(acc.at[sl], hbm0.at[sl])
    @pl.when(c == 1)
    def _():
      pltpu.sync_copy(acc.at[sl], hbm1.at[sl])
  _sliced_loop(s, wb)


GDEPTH = 5
N_GRP = N_ECH // GDEPTH


def _deg_pass(base, dst_hbm, acc, dst_vs, ones_v, psems, ssems):
  def dst_load(j, b):
    off = jnp.minimum(base + j * ECHUNK, N_EDGES - ECHUNK)
    return pltpu.async_copy(dst_hbm.at[pl.ds(off, ECHUNK)], dst_vs[b], psems[b])

  def scatter(b):
    return pltpu.async_copy(ones_v, acc.at[dst_vs[b]], ssems[b], add=True)

  d_descs = [dst_load(b, b) for b in range(GDEPTH)]

  def group(g, _):
    s_descs = []
    for b in range(GDEPTH):
      pltpu.make_async_copy(
          dst_hbm.at[pl.ds(base, ECHUNK)], dst_vs[b], psems[b]).wait()
      s_descs.append(scatter(b))
    for b in range(GDEPTH):
      s_descs[b].wait()
      dst_load((g + 1) * GDEPTH + b, b)
    return _
  lax.fori_loop(0, N_GRP - 1, group, None)

  s_descs = []
  for b in range(GDEPTH):
    pltpu.make_async_copy(
        dst_hbm.at[pl.ds(base, ECHUNK)], dst_vs[b], psems[b]).wait()
    s_descs.append(scatter(b))
  for b in range(GDEPTH):
    s_descs[b].wait()


def _rows_pass(base, src_hbm, dst_hbm, tab_hbm, acc,
               idx_vs, dst_vs, rows_vs, psems, gsems, ssems):
  def idx_load(j, b):
    off = jnp.minimum(base + j * ECHUNK, N_EDGES - ECHUNK)
    return (pltpu.async_copy(src_hbm.at[pl.ds(off, ECHUNK)], idx_vs[b], psems[b]),
            pltpu.async_copy(dst_hbm.at[pl.ds(off, ECHUNK)], dst_vs[b], psems[b]))

  def gather(b):
    return pltpu.async_copy(tab_hbm.at[idx_vs[b]], rows_vs[b], gsems[b])

  def scatter(b):
    return pltpu.async_copy(rows_vs[b], acc.at[dst_vs[b]], ssems[b], add=True)

  p_descs = [idx_load(b, b) for b in range(GDEPTH)]
  g_descs = []
  for b in range(GDEPTH):
    p_descs[b][0].wait()
    p_descs[b][1].wait()
    g_descs.append(gather(b))

  def group(g, _):
    s_descs = []
    for b in range(GDEPTH):
      pltpu.make_async_copy(tab_hbm.at[idx_vs[b]], rows_vs[b], gsems[b]).wait()
      s_descs.append(scatter(b))
    i_next = []
    for b in range(GDEPTH):
      s_descs[b].wait()
      i_next.append(idx_load((g + 1) * GDEPTH + b, b))
    for b in range(GDEPTH):
      i_next[b][0].wait()
      i_next[b][1].wait()
      gather(b)
    return _
  lax.fori_loop(0, N_GRP - 1, group, None)

  s_descs = []
  for b in range(GDEPTH):
    pltpu.make_async_copy(tab_hbm.at[idx_vs[b]], rows_vs[b], gsems[b]).wait()
    s_descs.append(scatter(b))
  for b in range(GDEPTH):
    s_descs[b].wait()


def _hop1_body(src_hbm, dst_hbm, tab_hbm,
               p0_hbm, p1_hbm, d0_hbm, d1_hbm, acc, *rest):
  idx_vs = rest[0:GDEPTH]
  dst_vs = rest[GDEPTH:2 * GDEPTH]
  rows_vs = rest[2 * GDEPTH:3 * GDEPTH]
  ones_v, zbuf = rest[3 * GDEPTH:3 * GDEPTH + 2]
  sems = rest[3 * GDEPTH + 2:]
  psems, gsems, ssems = (sems[0:GDEPTH], sems[GDEPTH:2 * GDEPTH],
                         sems[2 * GDEPTH:3 * GDEPTH])
  c = lax.axis_index("c")
  s = lax.axis_index("s")
  wid = s * NC + c
  base = wid * E_PER_W

  _fill(zbuf, RBLK, 0.0)
  _fill(ones_v, ECHUNK, 1.0)

  _zero_acc(s, acc, zbuf)
  plsc.subcore_barrier()
  _deg_pass(base, dst_hbm, acc, dst_vs, ones_v, psems, ssems)
  plsc.subcore_barrier()
  _wb_acc(c, s, acc, d0_hbm, d1_hbm)

  _zero_acc(s, acc, zbuf)
  plsc.subcore_barrier()
  _rows_pass(base, src_hbm, dst_hbm, tab_hbm, acc,
             idx_vs, dst_vs, rows_vs, psems, gsems, ssems)
  plsc.subcore_barrier()
  _wb_acc(c, s, acc, p0_hbm, p1_hbm)


def _hop2_body(src_hbm, dst_hbm, tab_hbm, p0_hbm, p1_hbm, acc, *rest):
  idx_vs = rest[0:GDEPTH]
  dst_vs = rest[GDEPTH:2 * GDEPTH]
  rows_vs = rest[2 * GDEPTH:3 * GDEPTH]
  zbuf = rest[3 * GDEPTH]
  sems = rest[3 * GDEPTH + 1:]
  psems, gsems, ssems = (sems[0:GDEPTH], sems[GDEPTH:2 * GDEPTH],
                         sems[2 * GDEPTH:3 * GDEPTH])
  c = lax.axis_index("c")
  s = lax.axis_index("s")
  wid = s * NC + c
  base = wid * E_PER_W

  _fill(zbuf, RBLK, 0.0)
  _zero_acc(s, acc, zbuf)
  plsc.subcore_barrier()
  _rows_pass(base, src_hbm, dst_hbm, tab_hbm, acc,
             idx_vs, dst_vs, rows_vs, psems, gsems, ssems)
  plsc.subcore_barrier()
  _wb_acc(c, s, acc, p0_hbm, p1_hbm)


_nd = jax.ShapeDtypeStruct((N_NODES, D), _f32)

_hop1 = pl.kernel(
    _hop1_body,
    out_type=(_nd, _nd, _nd, _nd),
    mesh=_mesh,
    scratch_types=(
        (pltpu.VMEM_SHARED((N_NODES, D), _f32),)
        + tuple(pltpu.VMEM((ECHUNK,), jnp.int32) for _ in range(GDEPTH))
        + tuple(pltpu.VMEM((ECHUNK,), jnp.int32) for _ in range(GDEPTH))
        + tuple(pltpu.VMEM((ECHUNK, D), _f32) for _ in range(GDEPTH))
        + (pltpu.VMEM((ECHUNK, D), _f32), pltpu.VMEM((RBLK, D), _f32))
        + tuple(pltpu.SemaphoreType.DMA for _ in range(3 * GDEPTH))))

_hop2 = pl.kernel(
    _hop2_body,
    out_type=(_nd, _nd),
    mesh=_mesh,
    scratch_types=(
        (pltpu.VMEM_SHARED((N_NODES, D), _f32),)
        + tuple(pltpu.VMEM((ECHUNK,), jnp.int32) for _ in range(GDEPTH))
        + tuple(pltpu.VMEM((ECHUNK,), jnp.int32) for _ in range(GDEPTH))
        + tuple(pltpu.VMEM((ECHUNK, D), _f32) for _ in range(GDEPTH))
        + (pltpu.VMEM((RBLK, D), _f32),)
        + tuple(pltpu.SemaphoreType.DMA for _ in range(3 * GDEPTH))))


def _combine_body(p0, p1, d0h, d1h, x1_hbm, rdeg_hbm, *rest):
  bufs = [rest[0:5], rest[5:10]]
  lsems = [rest[10:14], rest[14:18]]
  wsems = [rest[18:20], rest[20:22]]
  c = lax.axis_index("c")
  s = lax.axis_index("s")
  wid = s * NC + c

  def load(blk, bs, ls):
    sl = pl.ds(blk * RBLK, RBLK)
    return [pltpu.async_copy(src.at[sl], dst, sem)
            for src, dst, sem in zip((p0, p1, d0h, d1h), bs[:4], ls)]

  def compute_store(blk, bs, ws):
    b0, b1, db0, db1, rbuf = bs
    sl = pl.ds(blk * RBLK, RBLK)
    for r in range(RBLK):
      degv = db0[r, pl.ds(0, L)] + db1[r, pl.ds(0, L)]
      rv = 1.0 / jnp.maximum(degv, 1.0)
      rs = rv[0]
      for q in range(D // L):
        qsl = pl.ds(q * L, L)
        rbuf[r, qsl] = jnp.broadcast_to(rs, (L,))
        b0[r, qsl] = (b0[r, qsl] + b1[r, qsl]) * rs
    return [pltpu.async_copy(b0, x1_hbm.at[sl], ws[0]),
            pltpu.async_copy(rbuf, rdeg_hbm.at[sl], ws[1])]

  def pair(k2, _):
    blk_a = wid + NW * (2 * k2)
    blk_b = wid + NW * (2 * k2 + 1)

    @pl.when(blk_a < N_RBLK)
    def _():
      la = load(blk_a, bufs[0], lsems[0])

      @pl.when(blk_b < N_RBLK)
      def _():
        lb = load(blk_b, bufs[1], lsems[1])
        for d in la:
          d.wait()
        wa = compute_store(blk_a, bufs[0], wsems[0])
        for d in lb:
          d.wait()
        wb = compute_store(blk_b, bufs[1], wsems[1])
        for d in wa + wb:
          d.wait()

      @pl.when(blk_b >= N_RBLK)
      def _():
        for d in la:
          d.wait()
        wa = compute_store(blk_a, bufs[0], wsems[0])
        for d in wa:
          d.wait()
    return _
  lax.fori_loop(0, (N_RBLK + 2 * NW - 1) // (2 * NW), pair, None)


_combine = pl.kernel(
    _combine_body,
    out_type=(_nd, _nd),
    mesh=_mesh,
    scratch_types=(
        tuple(pltpu.VMEM((RBLK, D), _f32) for _ in range(10))
        + tuple(pltpu.SemaphoreType.DMA for _ in range(12))))


def _final_body(q0, q1, rdeg_hbm, tab_hbm, ids_hbm, f0_hbm, f1_hbm,
                idx_v, b0, b1, dd, fb, sem):
  c = lax.axis_index("c")
  s = lax.axis_index("s")
  wid = s * NC + c
  base = wid * B_PER_W

  pltpu.sync_copy(ids_hbm.at[pl.ds(base, B_PER_W)], idx_v)
  pltpu.async_copy(q0.at[idx_v], b0, sem).wait()
  pltpu.async_copy(q1.at[idx_v], b1, sem).wait()
  pltpu.async_copy(rdeg_hbm.at[idx_v], dd, sem).wait()
  pltpu.async_copy(tab_hbm.at[idx_v], fb, sem).wait()

  def row_body(r, _):
    rs = dd[r, pl.ds(0, L)][0]
    for q in range(D // L):
      qsl = pl.ds(q * L, L)
      b0[r, qsl] = (b0[r, qsl] + b1[r, qsl]) * rs
    return _
  lax.fori_loop(0, B_PER_W, row_body, None)

  pltpu.sync_copy(b0, f0_hbm.at[pl.ds(base, B_PER_W)])
  pltpu.sync_copy(fb, f1_hbm.at[pl.ds(base, B_PER_W)])


_final = pl.kernel(
    _final_body,
    out_type=(jax.ShapeDtypeStruct((B, D), _f32),
              jax.ShapeDtypeStruct((B, D), _f32)),
    mesh=_mesh,
    scratch_types=(pltpu.VMEM((B_PER_W,), jnp.int32),
                   pltpu.VMEM((B_PER_W, D), _f32),
                   pltpu.VMEM((B_PER_W, D), _f32),
                   pltpu.VMEM((B_PER_W, D), _f32),
                   pltpu.VMEM((B_PER_W, D), _f32),
                   pltpu.SemaphoreType.DMA))


TC_BLK = 512


def _tc_body(f0_ref, f1_ref, w_ref, b_ref, o_ref):
  acc = jnp.dot(f0_ref[...], w_ref[...], preferred_element_type=_f32)
  sc = jnp.dot(f1_ref[...], b_ref[...], preferred_element_type=_f32)
  o_ref[...] = jnp.maximum(acc + sc, 0.0)


_tc_epilogue = pl.pallas_call(
    _tc_body,
    grid=(B // TC_BLK,),
    in_specs=[
        pl.BlockSpec((TC_BLK, D), lambda i: (i, 0)),
        pl.BlockSpec((TC_BLK, D), lambda i: (i, 0)),
        pl.BlockSpec((D, D), lambda i: (0, 0)),
        pl.BlockSpec((D, 1), lambda i: (0, 0)),
    ],
    out_specs=pl.BlockSpec((TC_BLK, D), lambda i: (i, 0)),
    out_shape=jax.ShapeDtypeStruct((B, D), _f32),
)


def kernel(inputs, edge_index, table, weight, bias):
  ids = inputs.astype(jnp.int32)
  src = edge_index[0].astype(jnp.int32)
  dst = edge_index[1].astype(jnp.int32)
  table = table.astype(_f32)

  p0, p1, d0, d1 = _hop1(src, dst, table)
  x1, rdeg = _combine(p0, p1, d0, d1)
  q0, q1 = _hop2(src, dst, x1)
  f0, f1 = _final(q0, q1, rdeg, table, ids)
  return _tc_epilogue(f0, f1, weight, bias.reshape(D, 1))


# speedup vs baseline: 8.5997x; 1.1529x over previous
import jax
import jax.numpy as jnp
from jax import lax
from jax.experimental import pallas as pl
from jax.experimental.pallas import tpu as pltpu
from jax.experimental.pallas import tpu_sc as plsc

NC, NS, L = 2, 16, 16
NW = NC * NS
N_NODES = 10000
D = 128
N_EDGES = 320000
B = 4096

E_PER_W = N_EDGES // NW
ECHUNK = 40
N_ECH = E_PER_W // ECHUNK
RBLK = 16
N_PAD = 10240
N_RBLK = N_PAD // RBLK
B_PER_W = B // NW

_mesh = plsc.VectorSubcoreMesh(
    core_axis_name="c", subcore_axis_name="s", num_cores=NC, num_subcores=NS)

_f32 = jnp.float32


def _fill(buf, rows, value):
  v = jnp.full((L,), value, _f32)
  for r in range(rows):
    for q in range(D // L):
      buf[r, pl.ds(q * L, L)] = v


KD = 5


def _sliced_async(s, issue, sems):
  n_m = N_RBLK // NS

  def step(k, _):
    descs = [issue(s + NS * (KD * k + i), sems[i]) for i in range(KD)]
    for d in descs:
      d.wait()
    return _
  lax.fori_loop(0, n_m // KD, step, None)


def _zero_acc(s, acc, zbuf, sems):
  _sliced_async(s, lambda blk, sem: pltpu.async_copy(
      zbuf, acc.at[pl.ds(blk * RBLK, RBLK)], sem), sems)


def _wb_acc(c, s, acc, hbm0, hbm1, sems):
  @pl.when(c == 0)
  def _():
    _sliced_async(s, lambda blk, sem: pltpu.async_copy(
        acc.at[pl.ds(blk * RBLK, RBLK)],
        hbm0.at[pl.ds(blk * RBLK, RBLK)], sem), sems)
  @pl.when(c == 1)
  def _():
    _sliced_async(s, lambda blk, sem: pltpu.async_copy(
        acc.at[pl.ds(blk * RBLK, RBLK)],
        hbm1.at[pl.ds(blk * RBLK, RBLK)], sem), sems)


GDEPTH = 5
N_GRP = N_ECH // GDEPTH


def _deg_pass(base, dst_hbm, acc, dst_vs, ones_v, psems, ssems):
  def dst_load(j, b):
    off = jnp.minimum(base + j * ECHUNK, N_EDGES - ECHUNK)
    return pltpu.async_copy(dst_hbm.at[pl.ds(off, ECHUNK)], dst_vs[b], psems[b])

  def scatter(b):
    return pltpu.async_copy(ones_v, acc.at[dst_vs[b]], ssems[b], add=True)

  d_descs = [dst_load(b, b) for b in range(GDEPTH)]

  def group(g, _):
    s_descs = []
    for b in range(GDEPTH):
      pltpu.make_async_copy(
          dst_hbm.at[pl.ds(base, ECHUNK)], dst_vs[b], psems[b]).wait()
      s_descs.append(scatter(b))
    for b in range(GDEPTH):
      s_descs[b].wait()
      dst_load((g + 1) * GDEPTH + b, b)
    return _
  lax.fori_loop(0, N_GRP - 1, group, None)

  s_descs = []
  for b in range(GDEPTH):
    pltpu.make_async_copy(
        dst_hbm.at[pl.ds(base, ECHUNK)], dst_vs[b], psems[b]).wait()
    s_descs.append(scatter(b))
  for b in range(GDEPTH):
    s_descs[b].wait()


def _rows_pass(base, src_hbm, dst_hbm, tab_hbm, acc,
               idx_vs, dst_vs, rows_vs, psems, gsems, ssems):
  def idx_load(j, b):
    off = jnp.minimum(base + j * ECHUNK, N_EDGES - ECHUNK)
    return (pltpu.async_copy(src_hbm.at[pl.ds(off, ECHUNK)], idx_vs[b], psems[b]),
            pltpu.async_copy(dst_hbm.at[pl.ds(off, ECHUNK)], dst_vs[b], psems[b]))

  def gather(b):
    return pltpu.async_copy(tab_hbm.at[idx_vs[b]], rows_vs[b], gsems[b])

  def scatter(b):
    return pltpu.async_copy(rows_vs[b], acc.at[dst_vs[b]], ssems[b], add=True)

  p_descs = [idx_load(b, b) for b in range(GDEPTH)]
  g_descs = []
  for b in range(GDEPTH):
    p_descs[b][0].wait()
    p_descs[b][1].wait()
    g_descs.append(gather(b))

  def group(g, _):
    s_descs = []
    for b in range(GDEPTH):
      pltpu.make_async_copy(tab_hbm.at[idx_vs[b]], rows_vs[b], gsems[b]).wait()
      s_descs.append(scatter(b))
    i_next = []
    for b in range(GDEPTH):
      s_descs[b].wait()
      i_next.append(idx_load((g + 1) * GDEPTH + b, b))
    for b in range(GDEPTH):
      i_next[b][0].wait()
      i_next[b][1].wait()
      gather(b)
    return _
  lax.fori_loop(0, N_GRP - 1, group, None)

  s_descs = []
  for b in range(GDEPTH):
    pltpu.make_async_copy(tab_hbm.at[idx_vs[b]], rows_vs[b], gsems[b]).wait()
    s_descs.append(scatter(b))
  for b in range(GDEPTH):
    s_descs[b].wait()


def _hop1_body(src_hbm, dst_hbm, tab_hbm,
               p0_hbm, p1_hbm, d0_hbm, d1_hbm, acc, *rest):
  idx_vs = rest[0:GDEPTH]
  dst_vs = rest[GDEPTH:2 * GDEPTH]
  rows_vs = rest[2 * GDEPTH:3 * GDEPTH]
  ones_v, zbuf = rest[3 * GDEPTH:3 * GDEPTH + 2]
  sems = rest[3 * GDEPTH + 2:]
  psems, gsems, ssems = (sems[0:GDEPTH], sems[GDEPTH:2 * GDEPTH],
                         sems[2 * GDEPTH:3 * GDEPTH])
  c = lax.axis_index("c")
  s = lax.axis_index("s")
  wid = s * NC + c
  base = wid * E_PER_W

  _fill(zbuf, RBLK, 0.0)
  _fill(ones_v, ECHUNK, 1.0)

  _zero_acc(s, acc, zbuf, psems)
  plsc.subcore_barrier()
  _deg_pass(base, dst_hbm, acc, dst_vs, ones_v, psems, ssems)
  plsc.subcore_barrier()
  _wb_acc(c, s, acc, d0_hbm, d1_hbm, psems)

  _zero_acc(s, acc, zbuf, gsems)
  plsc.subcore_barrier()
  _rows_pass(base, src_hbm, dst_hbm, tab_hbm, acc,
             idx_vs, dst_vs, rows_vs, psems, gsems, ssems)
  plsc.subcore_barrier()
  _wb_acc(c, s, acc, p0_hbm, p1_hbm, psems)


def _hop2_body(src_hbm, dst_hbm, tab_hbm, p0_hbm, p1_hbm, acc, *rest):
  idx_vs = rest[0:GDEPTH]
  dst_vs = rest[GDEPTH:2 * GDEPTH]
  rows_vs = rest[2 * GDEPTH:3 * GDEPTH]
  zbuf = rest[3 * GDEPTH]
  sems = rest[3 * GDEPTH + 1:]
  psems, gsems, ssems = (sems[0:GDEPTH], sems[GDEPTH:2 * GDEPTH],
                         sems[2 * GDEPTH:3 * GDEPTH])
  c = lax.axis_index("c")
  s = lax.axis_index("s")
  wid = s * NC + c
  base = wid * E_PER_W

  _fill(zbuf, RBLK, 0.0)
  _zero_acc(s, acc, zbuf, psems)
  plsc.subcore_barrier()
  _rows_pass(base, src_hbm, dst_hbm, tab_hbm, acc,
             idx_vs, dst_vs, rows_vs, psems, gsems, ssems)
  plsc.subcore_barrier()
  _wb_acc(c, s, acc, p0_hbm, p1_hbm, psems)


_nd = jax.ShapeDtypeStruct((N_PAD, D), _f32)

_hop1 = pl.kernel(
    _hop1_body,
    out_type=(_nd, _nd, _nd, _nd),
    mesh=_mesh,
    scratch_types=(
        (pltpu.VMEM_SHARED((N_PAD, D), _f32),)
        + tuple(pltpu.VMEM((ECHUNK,), jnp.int32) for _ in range(GDEPTH))
        + tuple(pltpu.VMEM((ECHUNK,), jnp.int32) for _ in range(GDEPTH))
        + tuple(pltpu.VMEM((ECHUNK, D), _f32) for _ in range(GDEPTH))
        + (pltpu.VMEM((ECHUNK, D), _f32), pltpu.VMEM((RBLK, D), _f32))
        + tuple(pltpu.SemaphoreType.DMA for _ in range(3 * GDEPTH))))

_hop2 = pl.kernel(
    _hop2_body,
    out_type=(_nd, _nd),
    mesh=_mesh,
    scratch_types=(
        (pltpu.VMEM_SHARED((N_PAD, D), _f32),)
        + tuple(pltpu.VMEM((ECHUNK,), jnp.int32) for _ in range(GDEPTH))
        + tuple(pltpu.VMEM((ECHUNK,), jnp.int32) for _ in range(GDEPTH))
        + tuple(pltpu.VMEM((ECHUNK, D), _f32) for _ in range(GDEPTH))
        + (pltpu.VMEM((RBLK, D), _f32),)
        + tuple(pltpu.SemaphoreType.DMA for _ in range(3 * GDEPTH))))


def _combine_body(p0, p1, d0h, d1h, x1_hbm, rdeg_hbm, *rest):
  bufs = [rest[0:5], rest[5:10]]
  lsems = [rest[10:14], rest[14:18]]
  wsems = [rest[18:20], rest[20:22]]
  c = lax.axis_index("c")
  s = lax.axis_index("s")
  wid = s * NC + c

  def load(blk, bs, ls):
    sl = pl.ds(blk * RBLK, RBLK)
    return [pltpu.async_copy(src.at[sl], dst, sem)
            for src, dst, sem in zip((p0, p1, d0h, d1h), bs[:4], ls)]

  def compute_store(blk, bs, ws):
    b0, b1, db0, db1, rbuf = bs
    sl = pl.ds(blk * RBLK, RBLK)
    for r in range(RBLK):
      degv = db0[r, pl.ds(0, L)] + db1[r, pl.ds(0, L)]
      rv = 1.0 / jnp.maximum(degv, 1.0)
      rs = rv[0]
      for q in range(D // L):
        qsl = pl.ds(q * L, L)
        rbuf[r, qsl] = jnp.broadcast_to(rs, (L,))
        b0[r, qsl] = (b0[r, qsl] + b1[r, qsl]) * rs
    return [pltpu.async_copy(b0, x1_hbm.at[sl], ws[0]),
            pltpu.async_copy(rbuf, rdeg_hbm.at[sl], ws[1])]

  def pair(k2, _):
    blk_a = wid + NW * (2 * k2)
    blk_b = wid + NW * (2 * k2 + 1)

    @pl.when(blk_a < N_RBLK)
    def _():
      la = load(blk_a, bufs[0], lsems[0])

      @pl.when(blk_b < N_RBLK)
      def _():
        lb = load(blk_b, bufs[1], lsems[1])
        for d in la:
          d.wait()
        wa = compute_store(blk_a, bufs[0], wsems[0])
        for d in lb:
          d.wait()
        wb = compute_store(blk_b, bufs[1], wsems[1])
        for d in wa + wb:
          d.wait()

      @pl.when(blk_b >= N_RBLK)
      def _():
        for d in la:
          d.wait()
        wa = compute_store(blk_a, bufs[0], wsems[0])
        for d in wa:
          d.wait()
    return _
  lax.fori_loop(0, (N_RBLK + 2 * NW - 1) // (2 * NW), pair, None)


_combine = pl.kernel(
    _combine_body,
    out_type=(_nd, _nd),
    mesh=_mesh,
    scratch_types=(
        tuple(pltpu.VMEM((RBLK, D), _f32) for _ in range(10))
        + tuple(pltpu.SemaphoreType.DMA for _ in range(12))))


def _final_body(q0, q1, rdeg_hbm, tab_hbm, ids_hbm, f0_hbm, f1_hbm,
                idx_v, b0, b1, dd, fb, sem):
  c = lax.axis_index("c")
  s = lax.axis_index("s")
  wid = s * NC + c
  base = wid * B_PER_W

  pltpu.sync_copy(ids_hbm.at[pl.ds(base, B_PER_W)], idx_v)
  pltpu.async_copy(q0.at[idx_v], b0, sem).wait()
  pltpu.async_copy(q1.at[idx_v], b1, sem).wait()
  pltpu.async_copy(rdeg_hbm.at[idx_v], dd, sem).wait()
  pltpu.async_copy(tab_hbm.at[idx_v], fb, sem).wait()

  def row_body(r, _):
    rs = dd[r, pl.ds(0, L)][0]
    for q in range(D // L):
      qsl = pl.ds(q * L, L)
      b0[r, qsl] = (b0[r, qsl] + b1[r, qsl]) * rs
    return _
  lax.fori_loop(0, B_PER_W, row_body, None)

  pltpu.sync_copy(b0, f0_hbm.at[pl.ds(base, B_PER_W)])
  pltpu.sync_copy(fb, f1_hbm.at[pl.ds(base, B_PER_W)])


_final = pl.kernel(
    _final_body,
    out_type=(jax.ShapeDtypeStruct((B, D), _f32),
              jax.ShapeDtypeStruct((B, D), _f32)),
    mesh=_mesh,
    scratch_types=(pltpu.VMEM((B_PER_W,), jnp.int32),
                   pltpu.VMEM((B_PER_W, D), _f32),
                   pltpu.VMEM((B_PER_W, D), _f32),
                   pltpu.VMEM((B_PER_W, D), _f32),
                   pltpu.VMEM((B_PER_W, D), _f32),
                   pltpu.SemaphoreType.DMA))


TC_BLK = 512


def _tc_body(f0_ref, f1_ref, w_ref, b_ref, o_ref):
  acc = jnp.dot(f0_ref[...], w_ref[...], preferred_element_type=_f32)
  sc = jnp.dot(f1_ref[...], b_ref[...], preferred_element_type=_f32)
  o_ref[...] = jnp.maximum(acc + sc, 0.0)


_tc_epilogue = pl.pallas_call(
    _tc_body,
    grid=(B // TC_BLK,),
    in_specs=[
        pl.BlockSpec((TC_BLK, D), lambda i: (i, 0)),
        pl.BlockSpec((TC_BLK, D), lambda i: (i, 0)),
        pl.BlockSpec((D, D), lambda i: (0, 0)),
        pl.BlockSpec((D, 1), lambda i: (0, 0)),
    ],
    out_specs=pl.BlockSpec((TC_BLK, D), lambda i: (i, 0)),
    out_shape=jax.ShapeDtypeStruct((B, D), _f32),
)


def kernel(inputs, edge_index, table, weight, bias):
  ids = inputs.astype(jnp.int32)
  src = edge_index[0].astype(jnp.int32)
  dst = edge_index[1].astype(jnp.int32)
  table = table.astype(_f32)

  p0, p1, d0, d1 = _hop1(src, dst, table)
  x1, rdeg = _combine(p0, p1, d0, d1)
  q0, q1 = _hop2(src, dst, x1)
  f0, f1 = _final(q0, q1, rdeg, table, ids)
  return _tc_epilogue(f0, f1, weight, bias.reshape(D, 1))


# speedup vs baseline: 8.6296x; 1.0035x over previous
import jax
import jax.numpy as jnp
from jax import lax
from jax.experimental import pallas as pl
from jax.experimental.pallas import tpu as pltpu
from jax.experimental.pallas import tpu_sc as plsc

NC, NS, L = 2, 16, 16
NW = NC * NS
N_NODES = 10000
D = 128
N_EDGES = 320000
B = 4096

E_PER_W = N_EDGES // NW
ECHUNK = 40
N_ECH = E_PER_W // ECHUNK
RBLK = 16
N_PAD = 10240
N_RBLK = N_PAD // RBLK
B_PER_W = B // NW

_mesh = plsc.VectorSubcoreMesh(
    core_axis_name="c", subcore_axis_name="s", num_cores=NC, num_subcores=NS)

_f32 = jnp.float32


def _fill(buf, rows, value):
  v = jnp.full((L,), value, _f32)
  for r in range(rows):
    for q in range(D // L):
      buf[r, pl.ds(q * L, L)] = v


KD = 5


def _sliced_async(s, issue, sems):
  n_m = N_RBLK // NS

  def step(k, _):
    descs = [issue(s + NS * (KD * k + i), sems[i]) for i in range(KD)]
    for d in descs:
      d.wait()
    return _
  lax.fori_loop(0, n_m // KD, step, None)


def _zero_acc(s, acc, zbuf, sems):
  _sliced_async(s, lambda blk, sem: pltpu.async_copy(
      zbuf, acc.at[pl.ds(blk * RBLK, RBLK)], sem), sems)


def _wb_acc(c, s, acc, hbm0, hbm1, sems):
  @pl.when(c == 0)
  def _():
    _sliced_async(s, lambda blk, sem: pltpu.async_copy(
        acc.at[pl.ds(blk * RBLK, RBLK)],
        hbm0.at[pl.ds(blk * RBLK, RBLK)], sem), sems)
  @pl.when(c == 1)
  def _():
    _sliced_async(s, lambda blk, sem: pltpu.async_copy(
        acc.at[pl.ds(blk * RBLK, RBLK)],
        hbm1.at[pl.ds(blk * RBLK, RBLK)], sem), sems)


GDEPTH = 5
N_GRP = N_ECH // GDEPTH


def _deg_pass(base, dst_hbm, acc, dst_vs, ones_v, psems, ssems):
  def dst_load(j, b):
    off = jnp.minimum(base + j * ECHUNK, N_EDGES - ECHUNK)
    return pltpu.async_copy(dst_hbm.at[pl.ds(off, ECHUNK)], dst_vs[b], psems[b])

  def scatter(b):
    return pltpu.async_copy(ones_v, acc.at[dst_vs[b]], ssems[b], add=True)

  d_descs = [dst_load(b, b) for b in range(GDEPTH)]

  def group(g, _):
    s_descs = []
    for b in range(GDEPTH):
      pltpu.make_async_copy(
          dst_hbm.at[pl.ds(base, ECHUNK)], dst_vs[b], psems[b]).wait()
      s_descs.append(scatter(b))
    for b in range(GDEPTH):
      s_descs[b].wait()
      dst_load((g + 1) * GDEPTH + b, b)
    return _
  lax.fori_loop(0, N_GRP - 1, group, None)

  s_descs = []
  for b in range(GDEPTH):
    pltpu.make_async_copy(
        dst_hbm.at[pl.ds(base, ECHUNK)], dst_vs[b], psems[b]).wait()
    s_descs.append(scatter(b))
  for b in range(GDEPTH):
    s_descs[b].wait()


def _rows_pass(base, src_hbm, dst_hbm, tab_hbm, acc,
               idx_vs, dst_vs, rows_vs, psems, gsems, ssems):
  def idx_load(j, b):
    off = jnp.minimum(base + j * ECHUNK, N_EDGES - ECHUNK)
    return (pltpu.async_copy(src_hbm.at[pl.ds(off, ECHUNK)], idx_vs[b], psems[b]),
            pltpu.async_copy(dst_hbm.at[pl.ds(off, ECHUNK)], dst_vs[b], psems[b]))

  def gather(b):
    return pltpu.async_copy(tab_hbm.at[idx_vs[b]], rows_vs[b], gsems[b])

  def scatter(b):
    return pltpu.async_copy(rows_vs[b], acc.at[dst_vs[b]], ssems[b], add=True)

  p_descs = [idx_load(b, b) for b in range(GDEPTH)]
  g_descs = []
  for b in range(GDEPTH):
    p_descs[b][0].wait()
    p_descs[b][1].wait()
    g_descs.append(gather(b))

  def group(g, _):
    s_descs = []
    for b in range(GDEPTH):
      pltpu.make_async_copy(tab_hbm.at[idx_vs[b]], rows_vs[b], gsems[b]).wait()
      s_descs.append(scatter(b))
    i_next = []
    for b in range(GDEPTH):
      s_descs[b].wait()
      i_next.append(idx_load((g + 1) * GDEPTH + b, b))
    for b in range(GDEPTH):
      i_next[b][0].wait()
      i_next[b][1].wait()
      gather(b)
    return _
  lax.fori_loop(0, N_GRP - 1, group, None)

  s_descs = []
  for b in range(GDEPTH):
    pltpu.make_async_copy(tab_hbm.at[idx_vs[b]], rows_vs[b], gsems[b]).wait()
    s_descs.append(scatter(b))
  for b in range(GDEPTH):
    s_descs[b].wait()


def _hop1_body(src_hbm, dst_hbm, tab_hbm,
               p0_hbm, p1_hbm, d0_hbm, d1_hbm, acc, *rest):
  idx_vs = rest[0:GDEPTH]
  dst_vs = rest[GDEPTH:2 * GDEPTH]
  rows_vs = rest[2 * GDEPTH:3 * GDEPTH]
  ones_v, zbuf = rest[3 * GDEPTH:3 * GDEPTH + 2]
  sems = rest[3 * GDEPTH + 2:]
  psems, gsems, ssems = (sems[0:GDEPTH], sems[GDEPTH:2 * GDEPTH],
                         sems[2 * GDEPTH:3 * GDEPTH])
  c = lax.axis_index("c")
  s = lax.axis_index("s")
  wid = s * NC + c
  base = wid * E_PER_W

  _fill(zbuf, RBLK, 0.0)
  _fill(ones_v, ECHUNK, 1.0)

  _zero_acc(s, acc, zbuf, psems)
  plsc.subcore_barrier()
  _deg_pass(base, dst_hbm, acc, dst_vs, ones_v, psems, ssems)
  plsc.subcore_barrier()
  _wb_acc(c, s, acc, d0_hbm, d1_hbm, psems)

  _zero_acc(s, acc, zbuf, gsems)
  plsc.subcore_barrier()
  _rows_pass(base, src_hbm, dst_hbm, tab_hbm, acc,
             idx_vs, dst_vs, rows_vs, psems, gsems, ssems)
  plsc.subcore_barrier()
  _wb_acc(c, s, acc, p0_hbm, p1_hbm, psems)


def _hop2_body(src_hbm, dst_hbm, tab_hbm, p0_hbm, p1_hbm, acc, *rest):
  idx_vs = rest[0:GDEPTH]
  dst_vs = rest[GDEPTH:2 * GDEPTH]
  rows_vs = rest[2 * GDEPTH:3 * GDEPTH]
  zbuf = rest[3 * GDEPTH]
  sems = rest[3 * GDEPTH + 1:]
  psems, gsems, ssems = (sems[0:GDEPTH], sems[GDEPTH:2 * GDEPTH],
                         sems[2 * GDEPTH:3 * GDEPTH])
  c = lax.axis_index("c")
  s = lax.axis_index("s")
  wid = s * NC + c
  base = wid * E_PER_W

  _fill(zbuf, RBLK, 0.0)
  _zero_acc(s, acc, zbuf, psems)
  plsc.subcore_barrier()
  _rows_pass(base, src_hbm, dst_hbm, tab_hbm, acc,
             idx_vs, dst_vs, rows_vs, psems, gsems, ssems)
  plsc.subcore_barrier()
  _wb_acc(c, s, acc, p0_hbm, p1_hbm, psems)


_nd = jax.ShapeDtypeStruct((N_PAD, D), _f32)

_hop1 = pl.kernel(
    _hop1_body,
    out_type=(_nd, _nd, _nd, _nd),
    mesh=_mesh,
    scratch_types=(
        (pltpu.VMEM_SHARED((N_PAD, D), _f32),)
        + tuple(pltpu.VMEM((ECHUNK,), jnp.int32) for _ in range(GDEPTH))
        + tuple(pltpu.VMEM((ECHUNK,), jnp.int32) for _ in range(GDEPTH))
        + tuple(pltpu.VMEM((ECHUNK, D), _f32) for _ in range(GDEPTH))
        + (pltpu.VMEM((ECHUNK, D), _f32), pltpu.VMEM((RBLK, D), _f32))
        + tuple(pltpu.SemaphoreType.DMA for _ in range(3 * GDEPTH))))

_hop2 = pl.kernel(
    _hop2_body,
    out_type=(_nd, _nd),
    mesh=_mesh,
    scratch_types=(
        (pltpu.VMEM_SHARED((N_PAD, D), _f32),)
        + tuple(pltpu.VMEM((ECHUNK,), jnp.int32) for _ in range(GDEPTH))
        + tuple(pltpu.VMEM((ECHUNK,), jnp.int32) for _ in range(GDEPTH))
        + tuple(pltpu.VMEM((ECHUNK, D), _f32) for _ in range(GDEPTH))
        + (pltpu.VMEM((RBLK, D), _f32),)
        + tuple(pltpu.SemaphoreType.DMA for _ in range(3 * GDEPTH))))


def _combine_body(p0, p1, d0h, d1h, x1_hbm, rdeg_hbm, *rest):
  bufs = [rest[0:5], rest[5:10]]
  lsems = [rest[10:14], rest[14:18]]
  wsems = [rest[18:20], rest[20:22]]
  c = lax.axis_index("c")
  s = lax.axis_index("s")
  wid = s * NC + c

  def load(blk, bs, ls):
    sl = pl.ds(blk * RBLK, RBLK)
    return [pltpu.async_copy(src.at[sl], dst, sem)
            for src, dst, sem in zip((p0, p1, d0h, d1h), bs[:4], ls)]

  def compute_store(blk, bs, ws):
    b0, b1, db0, db1, rbuf = bs
    sl = pl.ds(blk * RBLK, RBLK)
    for r in range(RBLK):
      degv = db0[r, pl.ds(0, L)] + db1[r, pl.ds(0, L)]
      rv = 1.0 / jnp.maximum(degv, 1.0)
      rs = rv[0]
      for q in range(D // L):
        qsl = pl.ds(q * L, L)
        rbuf[r, qsl] = jnp.broadcast_to(rs, (L,))
        b0[r, qsl] = (b0[r, qsl] + b1[r, qsl]) * rs
    return [pltpu.async_copy(b0, x1_hbm.at[sl], ws[0]),
            pltpu.async_copy(rbuf, rdeg_hbm.at[sl], ws[1])]

  def pair(k2, _):
    blk_a = wid + NW * (2 * k2)
    blk_b = wid + NW * (2 * k2 + 1)

    @pl.when(blk_a < N_RBLK)
    def _():
      la = load(blk_a, bufs[0], lsems[0])

      @pl.when(blk_b < N_RBLK)
      def _():
        lb = load(blk_b, bufs[1], lsems[1])
        for d in la:
          d.wait()
        wa = compute_store(blk_a, bufs[0], wsems[0])
        for d in lb:
          d.wait()
        wb = compute_store(blk_b, bufs[1], wsems[1])
        for d in wa + wb:
          d.wait()

      @pl.when(blk_b >= N_RBLK)
      def _():
        for d in la:
          d.wait()
        wa = compute_store(blk_a, bufs[0], wsems[0])
        for d in wa:
          d.wait()
    return _
  lax.fori_loop(0, (N_RBLK + 2 * NW - 1) // (2 * NW), pair, None)


_combine = pl.kernel(
    _combine_body,
    out_type=(_nd, _nd),
    mesh=_mesh,
    scratch_types=(
        tuple(pltpu.VMEM((RBLK, D), _f32) for _ in range(10))
        + tuple(pltpu.SemaphoreType.DMA for _ in range(12))))


def _final_body(q0, q1, rdeg_hbm, tab_hbm, ids_hbm, f0_hbm, f1_hbm,
                idx_v, b0, b1, dd, fb, s0, s1, s2, s3):
  c = lax.axis_index("c")
  s = lax.axis_index("s")
  wid = s * NC + c
  base = wid * B_PER_W

  pltpu.sync_copy(ids_hbm.at[pl.ds(base, B_PER_W)], idx_v)
  descs = [pltpu.async_copy(q0.at[idx_v], b0, s0),
           pltpu.async_copy(q1.at[idx_v], b1, s1),
           pltpu.async_copy(rdeg_hbm.at[idx_v], dd, s2),
           pltpu.async_copy(tab_hbm.at[idx_v], fb, s3)]
  for d in descs:
    d.wait()

  def row_body(r, _):
    rs = dd[r, pl.ds(0, L)][0]
    for q in range(D // L):
      qsl = pl.ds(q * L, L)
      b0[r, qsl] = (b0[r, qsl] + b1[r, qsl]) * rs
    return _
  lax.fori_loop(0, B_PER_W, row_body, None)

  pltpu.sync_copy(b0, f0_hbm.at[pl.ds(base, B_PER_W)])
  pltpu.sync_copy(fb, f1_hbm.at[pl.ds(base, B_PER_W)])


_final = pl.kernel(
    _final_body,
    out_type=(jax.ShapeDtypeStruct((B, D), _f32),
              jax.ShapeDtypeStruct((B, D), _f32)),
    mesh=_mesh,
    scratch_types=(pltpu.VMEM((B_PER_W,), jnp.int32),
                   pltpu.VMEM((B_PER_W, D), _f32),
                   pltpu.VMEM((B_PER_W, D), _f32),
                   pltpu.VMEM((B_PER_W, D), _f32),
                   pltpu.VMEM((B_PER_W, D), _f32),
                   pltpu.SemaphoreType.DMA,
                   pltpu.SemaphoreType.DMA,
                   pltpu.SemaphoreType.DMA,
                   pltpu.SemaphoreType.DMA))


TC_BLK = 512


def _tc_body(f0_ref, f1_ref, w_ref, b_ref, o_ref):
  acc = jnp.dot(f0_ref[...], w_ref[...], preferred_element_type=_f32)
  sc = jnp.dot(f1_ref[...], b_ref[...], preferred_element_type=_f32)
  o_ref[...] = jnp.maximum(acc + sc, 0.0)


_tc_epilogue = pl.pallas_call(
    _tc_body,
    grid=(B // TC_BLK,),
    in_specs=[
        pl.BlockSpec((TC_BLK, D), lambda i: (i, 0)),
        pl.BlockSpec((TC_BLK, D), lambda i: (i, 0)),
        pl.BlockSpec((D, D), lambda i: (0, 0)),
        pl.BlockSpec((D, 1), lambda i: (0, 0)),
    ],
    out_specs=pl.BlockSpec((TC_BLK, D), lambda i: (i, 0)),
    out_shape=jax.ShapeDtypeStruct((B, D), _f32),
)


def kernel(inputs, edge_index, table, weight, bias):
  ids = inputs.astype(jnp.int32)
  src = edge_index[0].astype(jnp.int32)
  dst = edge_index[1].astype(jnp.int32)
  table = table.astype(_f32)

  p0, p1, d0, d1 = _hop1(src, dst, table)
  x1, rdeg = _combine(p0, p1, d0, d1)
  q0, q1 = _hop2(src, dst, x1)
  f0, f1 = _final(q0, q1, rdeg, table, ids)
  return _tc_epilogue(f0, f1, weight, bias.reshape(D, 1))
